# baseline (speedup 1.0000x reference)
import functools

import jax
import jax.numpy as jnp
from jax import lax
from jax.experimental import pallas as pl
from jax.experimental.pallas import tpu as pltpu
from jax.experimental.pallas import tpu_sc as plsc

_N = 10000
_E = 320000
_R = 5
_D = 32
_SEG = _N * _R
_NW = 32
_EW = _E // _NW
_CH = 125
_NCH = _EW // _CH
_CCOL = 16
_BLK = 1000
_NBLK = _SEG // _BLK
_ZB = 200
_NSLOT = 4
_QCH = _NCH // 4
_NB = 2000
_GN = _N // _NB


def _sc_mesh():
    return plsc.VectorSubcoreMesh(core_axis_name="c", subcore_axis_name="s")


@functools.partial(
    pl.kernel,
    mesh=_sc_mesh(),
    out_type=jax.ShapeDtypeStruct((2, _SEG, _CCOL), jnp.float32),
    compiler_params=pltpu.CompilerParams(use_tc_tiling_on_sc=False),
    scratch_types=[
        pltpu.VMEM((_NCH, _CH), jnp.int32),
        pltpu.VMEM((_CH, _CCOL), jnp.float32),
        pltpu.VMEM((_ZB, _CCOL), jnp.float32),
        pltpu.VMEM_SHARED((_SEG, _CCOL), jnp.float32),
        pltpu.SemaphoreType.DMA,
    ],
)
def _count_segments(seg_hbm, out_hbm, segv, onesv, zbuf, cnt_sh, sem):
    cid = lax.axis_index("c")
    sid = lax.axis_index("s")
    wid = sid * 2 + cid
    zeros16 = jnp.zeros((16,), jnp.float32)
    ones16 = jnp.ones((16,), jnp.float32)

    def _fill_z(j, carry):
        zbuf[j, :] = zeros16
        return carry

    lax.fori_loop(0, _ZB, _fill_z, 0)

    def _fill_o(j, carry):
        onesv[j, :] = ones16
        return carry

    lax.fori_loop(0, _CH, _fill_o, 0)

    for k in range((_NBLK + 15) // 16):
        b = sid + 16 * k

        @pl.when(b < _NBLK)
        def _():
            for m in range(_BLK // _ZB):
                pltpu.sync_copy(zbuf, cnt_sh.at[pl.ds(b * _BLK + m * _ZB, _ZB)])

    plsc.subcore_barrier()

    pltpu.sync_copy(seg_hbm.at[pl.ds(wid * _NCH, _NCH)], segv)

    def _scatter(j, carry):
        pltpu.sync_copy(onesv, cnt_sh.at[segv.at[j]], add=True)
        return carry

    lax.fori_loop(0, _NCH, _scatter, 0)
    plsc.subcore_barrier()

    for k in range((_NBLK + 15) // 16):
        b = sid + 16 * k

        @pl.when(b < _NBLK)
        def _():
            pltpu.sync_copy(cnt_sh.at[pl.ds(b * _BLK, _BLK)],
                            out_hbm.at[cid, pl.ds(b * _BLK, _BLK), :])


@functools.partial(
    pl.kernel,
    mesh=_sc_mesh(),
    out_type=jax.ShapeDtypeStruct((2, _SEG, _D), jnp.float32),
    compiler_params=pltpu.CompilerParams(use_tc_tiling_on_sc=False),
    scratch_types=[
        pltpu.VMEM((_QCH, _CH), jnp.int32),
        pltpu.VMEM((_QCH, _CH), jnp.int32),
        pltpu.VMEM((_NSLOT, _CH, _D), jnp.float32),
        pltpu.VMEM((_ZB, _D), jnp.float32),
        pltpu.VMEM_SHARED((_SEG, _D), jnp.float32),
        pltpu.SemaphoreType.DMA,
        pltpu.SemaphoreType.DMA,
        pltpu.SemaphoreType.DMA,
        pltpu.SemaphoreType.DMA,
    ],
)
def _segment_accumulate(hr_hbm, gidx_hbm, seg_hbm, out_hbm, gv, sv, rows, zbuf,
                        acc_sh, sg0, sg1, sg2, sg3):
    sems = [sg0, sg1, sg2, sg3]
    cid = lax.axis_index("c")
    sid = lax.axis_index("s")
    wid = sid * 2 + cid
    zeros16 = jnp.zeros((16,), jnp.float32)

    def _fill_z(j, carry):
        zbuf[j, pl.ds(0, 16)] = zeros16
        zbuf[j, pl.ds(16, 16)] = zeros16
        return carry

    lax.fori_loop(0, _ZB, _fill_z, 0)

    for k in range((_NBLK + 15) // 16):
        b = sid + 16 * k

        @pl.when(b < _NBLK)
        def _():
            for m in range(_BLK // _ZB):
                pltpu.sync_copy(zbuf, acc_sh.at[pl.ds(b * _BLK + m * _ZB, _ZB)])

    plsc.subcore_barrier()

    for q in range(4):
        pltpu.sync_copy(gidx_hbm.at[pl.ds(wid * _NCH + q * _QCH, _QCH)], gv)
        pltpu.sync_copy(seg_hbm.at[pl.ds(wid * _NCH + q * _QCH, _QCH)], sv)

        def _group(t, carry):
            handles = []
            for b in range(_NSLOT):
                jj = t * _NSLOT + b
                handles.append(pltpu.async_copy(
                    hr_hbm.at[gv.at[jj]], rows.at[b], sems[b]))
            for b in range(_NSLOT):
                jj = t * _NSLOT + b
                handles[b].wait()
                pltpu.sync_copy(rows.at[b], acc_sh.at[sv.at[jj]], add=True)
            return carry

        lax.fori_loop(0, _QCH // _NSLOT, _group, 0)
    plsc.subcore_barrier()

    for k in range((_NBLK + 15) // 16):
        b = sid + 16 * k

        @pl.when(b < _NBLK)
        def _():
            pltpu.sync_copy(acc_sh.at[pl.ds(b * _BLK, _BLK)],
                            out_hbm.at[cid, pl.ds(b * _BLK, _BLK), :])


def _prep_body(src_ref, dst_ref, et_ref, gidx_ref, seg_ref):
    et = et_ref[...]
    gidx_ref[...] = et * _N + src_ref[...]
    seg_ref[...] = et * _N + dst_ref[...]


def _icnt_body(cnt_ref, out_ref):
    cols = []
    for r in range(_R):
        c = cnt_ref[0, r, :, 0:1] + cnt_ref[1, r, :, 0:1]
        cols.append(1.0 / jnp.maximum(c, 1.0))
    out_ref[...] = jnp.concatenate(cols, axis=1)


def _wr(basis_ref, comp_ref, r):
    c = comp_ref[...]
    return basis_ref[0] * c[r:r + 1, 0:1] + basis_ref[1] * c[r:r + 1, 1:2]


def _hr0_body(x_ref, basis_ref, comp_ref, out_ref):
    x = x_ref[...]
    for r in range(_R):
        out_ref[r] = jnp.dot(x, _wr(basis_ref, comp_ref, r),
                             preferred_element_type=jnp.float32)


def _comb_core(s_ref, icnt_ref, h_ref, root_ref, bias_ref):
    agg = (s_ref[0, 0] + s_ref[1, 0]) * icnt_ref[:, 0:1]
    for r in range(1, _R):
        agg = agg + (s_ref[0, r] + s_ref[1, r]) * icnt_ref[:, r:r + 1]
    root_term = jnp.dot(h_ref[...], root_ref[...],
                        preferred_element_type=jnp.float32)
    return jnp.tanh(agg + root_term + bias_ref[...])


def _comb_mid_body(s_ref, icnt_ref, h_ref, root_ref, bias_ref,
                   basis_ref, comp_ref, hout_ref, hrn_ref):
    h = _comb_core(s_ref, icnt_ref, h_ref, root_ref, bias_ref)
    hout_ref[...] = h
    for r in range(_R):
        hrn_ref[r] = jnp.dot(h, _wr(basis_ref, comp_ref, r),
                             preferred_element_type=jnp.float32)


def _comb_last_body(s_ref, icnt_ref, h_ref, root_ref, bias_ref, hout_ref):
    hout_ref[...] = _comb_core(s_ref, icnt_ref, h_ref, root_ref, bias_ref)


def _final_body(h0_ref, h1_ref, h2_ref, h3_ref, bat_ref,
                w1_ref, b1_ref, w2_ref, b2_ref, out_ref, acc_ref):
    i = pl.program_id(0)

    @pl.when(i == 0)
    def _():
        acc_ref[...] = jnp.zeros_like(acc_ref)

    concat = jnp.concatenate(
        [h0_ref[...], h1_ref[...], h2_ref[...], h3_ref[...]], axis=1)
    g = lax.broadcasted_iota(jnp.int32, (128, _NB), 0)
    onehot = (bat_ref[0] == g).astype(jnp.float32)
    acc_ref[...] += jnp.dot(onehot, concat, preferred_element_type=jnp.float32)

    @pl.when(i == pl.num_programs(0) - 1)
    def _():
        pooled = acc_ref[...]
        hid = jnp.maximum(
            jnp.dot(pooled, w1_ref[...], preferred_element_type=jnp.float32)
            + b1_ref[...], 0.0)
        o = (jnp.dot(hid, w2_ref[...], preferred_element_type=jnp.float32)
             + b2_ref[...])
        m = jnp.max(o, axis=1, keepdims=True)
        lse = jnp.log(jnp.sum(jnp.exp(o - m), axis=1, keepdims=True)) + m
        out_ref[...] = o - lse


def kernel(x, edge_index, edge_type, batch,
           basis0, comp0, root0, bias0,
           basis1, comp1, root1, bias1,
           basis2, comp2, root2, bias2,
           basis3, comp3, root3, bias3,
           lin1_w, lin1_b, lin2_w, lin2_b):
    f32 = jnp.float32
    src2 = edge_index[0].reshape(2500, 128)
    dst2 = edge_index[1].reshape(2500, 128)
    et2 = edge_type.reshape(2500, 128)

    gidx2, seg2 = pl.pallas_call(
        _prep_body,
        out_shape=[jax.ShapeDtypeStruct((2500, 128), jnp.int32)] * 2,
    )(src2, dst2, et2)
    gidx = gidx2.reshape(_NW * _NCH, _CH)
    seg = seg2.reshape(_NW * _NCH, _CH)

    cnt2 = _count_segments(seg)
    icnt = pl.pallas_call(
        _icnt_body,
        grid=(_GN,),
        in_specs=[pl.BlockSpec((2, _R, _NB, _CCOL), lambda i: (0, 0, i, 0))],
        out_specs=pl.BlockSpec((_NB, _R), lambda i: (i, 0)),
        out_shape=jax.ShapeDtypeStruct((_N, _R), f32),
    )(cnt2.reshape(2, _R, _N, _CCOL))

    hr = pl.pallas_call(
        _hr0_body,
        grid=(_GN,),
        in_specs=[
            pl.BlockSpec((_NB, 128), lambda i: (i, 0)),
            pl.BlockSpec((2, 128, _D), lambda i: (0, 0, 0)),
            pl.BlockSpec((_R, 2), lambda i: (0, 0)),
        ],
        out_specs=pl.BlockSpec((_R, _NB, _D), lambda i: (0, i, 0)),
        out_shape=jax.ShapeDtypeStruct((_R, _N, _D), f32),
    )(x, basis0, comp0)

    layers = [
        (root0, bias0, basis1, comp1),
        (root1, bias1, basis2, comp2),
        (root2, bias2, basis3, comp3),
        (root3, bias3, None, None),
    ]
    h_in = x
    states = []
    for l, (root_l, bias_l, bnext, cnext) in enumerate(layers):
        s2 = _segment_accumulate(hr.reshape(_SEG, _D), gidx, seg)
        s2 = s2.reshape(2, _R, _N, _D)
        bias2d = bias_l.reshape(1, _D)
        din = h_in.shape[1]
        base_specs = [
            pl.BlockSpec((2, _R, _NB, _D), lambda i: (0, 0, i, 0)),
            pl.BlockSpec((_NB, _R), lambda i: (i, 0)),
            pl.BlockSpec((_NB, din), lambda i: (i, 0)),
            pl.BlockSpec((din, _D), lambda i: (0, 0)),
            pl.BlockSpec((1, _D), lambda i: (0, 0)),
        ]
        if bnext is not None:
            h, hr = pl.pallas_call(
                _comb_mid_body,
                grid=(_GN,),
                in_specs=base_specs + [
                    pl.BlockSpec((2, _D, _D), lambda i: (0, 0, 0)),
                    pl.BlockSpec((_R, 2), lambda i: (0, 0)),
                ],
                out_specs=[
                    pl.BlockSpec((_NB, _D), lambda i: (i, 0)),
                    pl.BlockSpec((_R, _NB, _D), lambda i: (0, i, 0)),
                ],
                out_shape=[
                    jax.ShapeDtypeStruct((_N, _D), f32),
                    jax.ShapeDtypeStruct((_R, _N, _D), f32),
                ],
            )(s2, icnt, h_in, root_l, bias2d, bnext, cnext)
        else:
            h = pl.pallas_call(
                _comb_last_body,
                grid=(_GN,),
                in_specs=base_specs,
                out_specs=pl.BlockSpec((_NB, _D), lambda i: (i, 0)),
                out_shape=jax.ShapeDtypeStruct((_N, _D), f32),
            )(s2, icnt, h_in, root_l, bias2d)
        states.append(h)
        h_in = h

    hspec = pl.BlockSpec((_NB, _D), lambda i: (i, 0))
    out = pl.pallas_call(
        _final_body,
        grid=(_GN,),
        in_specs=[hspec, hspec, hspec, hspec,
                  pl.BlockSpec((1, 1, _NB), lambda i: (i, 0, 0)),
                  pl.BlockSpec((128, 128), lambda i: (0, 0)),
                  pl.BlockSpec((1, 128), lambda i: (0, 0)),
                  pl.BlockSpec((128, 5), lambda i: (0, 0)),
                  pl.BlockSpec((1, 5), lambda i: (0, 0))],
        out_specs=pl.BlockSpec((128, 5), lambda i: (0, 0)),
        out_shape=jax.ShapeDtypeStruct((128, 5), f32),
        scratch_shapes=[pltpu.VMEM((128, 128), f32)],
    )(states[0], states[1], states[2], states[3], batch.reshape(_GN, 1, _NB),
      lin1_w, lin1_b.reshape(1, 128), lin2_w, lin2_b.reshape(1, 5))
    return out


# speedup vs baseline: 1.2377x; 1.2377x over previous
import functools

import jax
import jax.numpy as jnp
from jax import lax
from jax.experimental import pallas as pl
from jax.experimental.pallas import tpu as pltpu
from jax.experimental.pallas import tpu_sc as plsc

_N = 10000
_E = 320000
_R = 5
_D = 32
_SEG = _N * _R
_NW = 32
_EW = _E // _NW
_CH = 125
_NCH = _EW // _CH
_CCOL = 16
_BLK = 1000
_NBLK = _SEG // _BLK
_ZB = 200
_NSLOT = 4
_QCH = _NCH // 4
_NB = 2000
_GN = _N // _NB


def _sc_mesh():
    return plsc.VectorSubcoreMesh(core_axis_name="c", subcore_axis_name="s")


@functools.partial(
    pl.kernel,
    mesh=_sc_mesh(),
    out_type=jax.ShapeDtypeStruct((2, _SEG, _CCOL), jnp.float32),
    compiler_params=pltpu.CompilerParams(use_tc_tiling_on_sc=False),
    scratch_types=[
        pltpu.VMEM((_NCH, _CH), jnp.int32),
        pltpu.VMEM((_CH, _CCOL), jnp.float32),
        pltpu.VMEM((_ZB, _CCOL), jnp.float32),
        pltpu.VMEM_SHARED((_SEG, _CCOL), jnp.float32),
        pltpu.SemaphoreType.DMA,
    ],
)
def _count_segments(seg_hbm, out_hbm, segv, onesv, zbuf, cnt_sh, sem):
    cid = lax.axis_index("c")
    sid = lax.axis_index("s")
    wid = sid * 2 + cid
    zeros16 = jnp.zeros((16,), jnp.float32)
    ones16 = jnp.ones((16,), jnp.float32)

    def _fill_z(j, carry):
        zbuf[j, :] = zeros16
        return carry

    lax.fori_loop(0, _ZB, _fill_z, 0)

    def _fill_o(j, carry):
        onesv[j, :] = ones16
        return carry

    lax.fori_loop(0, _CH, _fill_o, 0)

    for k in range((_NBLK + 15) // 16):
        b = sid + 16 * k

        @pl.when(b < _NBLK)
        def _():
            for m in range(_BLK // _ZB):
                pltpu.sync_copy(zbuf, cnt_sh.at[pl.ds(b * _BLK + m * _ZB, _ZB)])

    plsc.subcore_barrier()

    pltpu.sync_copy(seg_hbm.at[pl.ds(wid * _NCH, _NCH)], segv)

    def _scatter(j, carry):
        pltpu.sync_copy(onesv, cnt_sh.at[segv.at[j]], add=True)
        return carry

    lax.fori_loop(0, _NCH, _scatter, 0)
    plsc.subcore_barrier()

    for k in range((_NBLK + 15) // 16):
        b = sid + 16 * k

        @pl.when(b < _NBLK)
        def _():
            pltpu.sync_copy(cnt_sh.at[pl.ds(b * _BLK, _BLK)],
                            out_hbm.at[cid, pl.ds(b * _BLK, _BLK), :])


@functools.partial(
    pl.kernel,
    mesh=_sc_mesh(),
    out_type=jax.ShapeDtypeStruct((2, _SEG, _D), jnp.float32),
    compiler_params=pltpu.CompilerParams(use_tc_tiling_on_sc=False),
    scratch_types=[
        pltpu.VMEM((_QCH, _CH), jnp.int32),
        pltpu.VMEM((_QCH, _CH), jnp.int32),
        pltpu.VMEM((_NSLOT, _CH, _D), jnp.float32),
        pltpu.VMEM((_ZB, _D), jnp.float32),
        pltpu.VMEM_SHARED((_SEG, _D), jnp.float32),
        pltpu.SemaphoreType.DMA,
        pltpu.SemaphoreType.DMA,
        pltpu.SemaphoreType.DMA,
        pltpu.SemaphoreType.DMA,
    ],
)
def _segment_accumulate(hr_hbm, gidx_hbm, seg_hbm, cnt_hbm, out_hbm, gv, sv,
                        rows, zbuf, acc_sh, sg0, sg1, sg2, sg3):
    del cnt_hbm
    sems = [sg0, sg1, sg2, sg3]
    cid = lax.axis_index("c")
    sid = lax.axis_index("s")
    wid = sid * 2 + cid
    zeros16 = jnp.zeros((16,), jnp.float32)

    def _fill_z(j, carry):
        zbuf[j, pl.ds(0, 16)] = zeros16
        zbuf[j, pl.ds(16, 16)] = zeros16
        return carry

    lax.fori_loop(0, _ZB, _fill_z, 0)

    for k in range((_NBLK + 15) // 16):
        b = sid + 16 * k

        @pl.when(b < _NBLK)
        def _():
            for m in range(_BLK // _ZB):
                pltpu.sync_copy(zbuf, acc_sh.at[pl.ds(b * _BLK + m * _ZB, _ZB)])

    plsc.subcore_barrier()

    for q in range(4):
        pltpu.sync_copy(gidx_hbm.at[pl.ds(wid * _NCH + q * _QCH, _QCH)], gv)
        pltpu.sync_copy(seg_hbm.at[pl.ds(wid * _NCH + q * _QCH, _QCH)], sv)

        def _group(t, carry):
            handles = []
            for b in range(_NSLOT):
                jj = t * _NSLOT + b
                handles.append(pltpu.async_copy(
                    hr_hbm.at[gv.at[jj]], rows.at[b], sems[b]))
            for b in range(_NSLOT):
                jj = t * _NSLOT + b
                handles[b].wait()
                pltpu.sync_copy(rows.at[b], acc_sh.at[sv.at[jj]], add=True)
            return carry

        lax.fori_loop(0, _QCH // _NSLOT, _group, 0)
    plsc.subcore_barrier()

    for k in range((_NBLK + 15) // 16):
        b = sid + 16 * k

        @pl.when(b < _NBLK)
        def _():
            pltpu.sync_copy(acc_sh.at[pl.ds(b * _BLK, _BLK)],
                            out_hbm.at[cid, pl.ds(b * _BLK, _BLK), :])


def _prep_body(src_ref, dst_ref, et_ref, gidx_ref, seg_ref):
    et = et_ref[...]
    gidx_ref[...] = src_ref[...] * _R + et
    seg_ref[...] = dst_ref[...] * _R + et


def _icnt_body(cnt_ref, out_ref):
    c0 = cnt_ref[0, :, 0:1]
    c1 = cnt_ref[1, :, 0:1]
    out_ref[...] = 1.0 / jnp.maximum(c0 + c1, 1.0)


def _wcat(basis_ref, comp_ref):
    c = comp_ref[...]
    b0 = basis_ref[0]
    b1 = basis_ref[1]
    cols = [b0 * c[r:r + 1, 0:1] + b1 * c[r:r + 1, 1:2] for r in range(_R)]
    return jnp.concatenate(cols, axis=1)


def _hr0_body(x_ref, basis_ref, comp_ref, out_ref):
    w = _wcat(basis_ref, comp_ref)
    out_ref[...] = jnp.dot(x_ref[...], w, preferred_element_type=jnp.float32)


def _comb_core(s_ref, icnt_ref, h_ref, root_ref, bias_ref):
    s = s_ref[0] + s_ref[1]
    agg = s[:, 0:_D] * icnt_ref[:, 0:1]
    for r in range(1, _R):
        agg = agg + s[:, _D * r:_D * (r + 1)] * icnt_ref[:, r:r + 1]
    root_term = jnp.dot(h_ref[...], root_ref[...],
                        preferred_element_type=jnp.float32)
    return jnp.tanh(agg + root_term + bias_ref[...])


def _comb_mid_body(s_ref, icnt_ref, h_ref, root_ref, bias_ref,
                   basis_ref, comp_ref, hout_ref, hrn_ref):
    h = _comb_core(s_ref, icnt_ref, h_ref, root_ref, bias_ref)
    hout_ref[...] = h
    w = _wcat(basis_ref, comp_ref)
    hrn_ref[...] = jnp.dot(h, w, preferred_element_type=jnp.float32)


def _comb_last_body(s_ref, icnt_ref, h_ref, root_ref, bias_ref, hout_ref):
    hout_ref[...] = _comb_core(s_ref, icnt_ref, h_ref, root_ref, bias_ref)


def _final_body(h0_ref, h1_ref, h2_ref, h3_ref, bat_ref,
                w1_ref, b1_ref, w2_ref, b2_ref, out_ref, acc_ref):
    i = pl.program_id(0)

    @pl.when(i == 0)
    def _():
        acc_ref[...] = jnp.zeros_like(acc_ref)

    concat = jnp.concatenate(
        [h0_ref[...], h1_ref[...], h2_ref[...], h3_ref[...]], axis=1)
    g = lax.broadcasted_iota(jnp.int32, (128, _NB), 0)
    onehot = (bat_ref[0] == g).astype(jnp.float32)
    acc_ref[...] += jnp.dot(onehot, concat, preferred_element_type=jnp.float32)

    @pl.when(i == pl.num_programs(0) - 1)
    def _():
        pooled = acc_ref[...]
        hid = jnp.maximum(
            jnp.dot(pooled, w1_ref[...], preferred_element_type=jnp.float32)
            + b1_ref[...], 0.0)
        o = (jnp.dot(hid, w2_ref[...], preferred_element_type=jnp.float32)
             + b2_ref[...])
        m = jnp.max(o, axis=1, keepdims=True)
        lse = jnp.log(jnp.sum(jnp.exp(o - m), axis=1, keepdims=True)) + m
        out_ref[...] = o - lse


def kernel(x, edge_index, edge_type, batch,
           basis0, comp0, root0, bias0,
           basis1, comp1, root1, bias1,
           basis2, comp2, root2, bias2,
           basis3, comp3, root3, bias3,
           lin1_w, lin1_b, lin2_w, lin2_b):
    f32 = jnp.float32
    src2 = edge_index[0].reshape(2500, 128)
    dst2 = edge_index[1].reshape(2500, 128)
    et2 = edge_type.reshape(2500, 128)

    gidx2, seg2 = pl.pallas_call(
        _prep_body,
        out_shape=[jax.ShapeDtypeStruct((2500, 128), jnp.int32)] * 2,
    )(src2, dst2, et2)
    gidx = gidx2.reshape(_NW * _NCH, _CH)
    seg = seg2.reshape(_NW * _NCH, _CH)

    cnt2 = _count_segments(seg)
    icnt = pl.pallas_call(
        _icnt_body,
        grid=(_NBLK,),
        in_specs=[pl.BlockSpec((2, _BLK, _CCOL), lambda i: (0, i, 0))],
        out_specs=pl.BlockSpec((_BLK, 1), lambda i: (i, 0)),
        out_shape=jax.ShapeDtypeStruct((_SEG, 1), f32),
    )(cnt2)
    icnt = icnt.reshape(_N, _R)

    hr = pl.pallas_call(
        _hr0_body,
        grid=(_GN,),
        in_specs=[
            pl.BlockSpec((_NB, 128), lambda i: (i, 0)),
            pl.BlockSpec((2, 128, _D), lambda i: (0, 0, 0)),
            pl.BlockSpec((_R, 2), lambda i: (0, 0)),
        ],
        out_specs=pl.BlockSpec((_NB, _R * _D), lambda i: (i, 0)),
        out_shape=jax.ShapeDtypeStruct((_N, _R * _D), f32),
    )(x, basis0, comp0)

    layers = [
        (root0, bias0, basis1, comp1),
        (root1, bias1, basis2, comp2),
        (root2, bias2, basis3, comp3),
        (root3, bias3, None, None),
    ]
    h_in = x
    states = []
    for l, (root_l, bias_l, bnext, cnext) in enumerate(layers):
        s2 = _segment_accumulate(hr.reshape(_SEG, _D), gidx, seg, cnt2)
        s2 = s2.reshape(2, _N, _R * _D)
        bias2d = bias_l.reshape(1, _D)
        din = h_in.shape[1]
        base_specs = [
            pl.BlockSpec((2, _NB, _R * _D), lambda i: (0, i, 0)),
            pl.BlockSpec((_NB, _R), lambda i: (i, 0)),
            pl.BlockSpec((_NB, din), lambda i: (i, 0)),
            pl.BlockSpec((din, _D), lambda i: (0, 0)),
            pl.BlockSpec((1, _D), lambda i: (0, 0)),
        ]
        if bnext is not None:
            h, hr = pl.pallas_call(
                _comb_mid_body,
                grid=(_GN,),
                in_specs=base_specs + [
                    pl.BlockSpec((2, _D, _D), lambda i: (0, 0, 0)),
                    pl.BlockSpec((_R, 2), lambda i: (0, 0)),
                ],
                out_specs=[
                    pl.BlockSpec((_NB, _D), lambda i: (i, 0)),
                    pl.BlockSpec((_NB, _R * _D), lambda i: (i, 0)),
                ],
                out_shape=[
                    jax.ShapeDtypeStruct((_N, _D), f32),
                    jax.ShapeDtypeStruct((_N, _R * _D), f32),
                ],
            )(s2, icnt, h_in, root_l, bias2d, bnext, cnext)
        else:
            h = pl.pallas_call(
                _comb_last_body,
                grid=(_GN,),
                in_specs=base_specs,
                out_specs=pl.BlockSpec((_NB, _D), lambda i: (i, 0)),
                out_shape=jax.ShapeDtypeStruct((_N, _D), f32),
            )(s2, icnt, h_in, root_l, bias2d)
        states.append(h)
        h_in = h

    hspec = pl.BlockSpec((_NB, _D), lambda i: (i, 0))
    out = pl.pallas_call(
        _final_body,
        grid=(_GN,),
        in_specs=[hspec, hspec, hspec, hspec,
                  pl.BlockSpec((1, 1, _NB), lambda i: (i, 0, 0)),
                  pl.BlockSpec((128, 128), lambda i: (0, 0)),
                  pl.BlockSpec((1, 128), lambda i: (0, 0)),
                  pl.BlockSpec((128, 5), lambda i: (0, 0)),
                  pl.BlockSpec((1, 5), lambda i: (0, 0))],
        out_specs=pl.BlockSpec((128, 5), lambda i: (0, 0)),
        out_shape=jax.ShapeDtypeStruct((128, 5), f32),
        scratch_shapes=[pltpu.VMEM((128, 128), f32)],
    )(states[0], states[1], states[2], states[3], batch.reshape(_GN, 1, _NB),
      lin1_w, lin1_b.reshape(1, 128), lin2_w, lin2_b.reshape(1, 5))
    return out


# speedup vs baseline: 1.3028x; 1.0526x over previous
import functools

import jax
import jax.numpy as jnp
from jax import lax
from jax.experimental import pallas as pl
from jax.experimental.pallas import tpu as pltpu
from jax.experimental.pallas import tpu_sc as plsc

_N = 10000
_E = 320000
_R = 5
_D = 32
_SEG = _N * _R
_NW = 32
_EW = _E // _NW
_CH = 125
_NCH = _EW // _CH
_CCOL = 16
_BLK = 1000
_NBLK = _SEG // _BLK
_ZB = 200
_NSLOT = 4
_QCH = _NCH // 4
_NB = 2000
_GN = _N // _NB


def _sc_mesh():
    return plsc.VectorSubcoreMesh(core_axis_name="c", subcore_axis_name="s")


@functools.partial(
    pl.kernel,
    mesh=_sc_mesh(),
    out_type=jax.ShapeDtypeStruct((2, _SEG, _CCOL), jnp.float32),
    compiler_params=pltpu.CompilerParams(use_tc_tiling_on_sc=False),
    scratch_types=[
        pltpu.VMEM((_NCH, _CH), jnp.int32),
        pltpu.VMEM((_CH, _CCOL), jnp.float32),
        pltpu.VMEM((_ZB, _CCOL), jnp.float32),
        pltpu.VMEM_SHARED((_SEG, _CCOL), jnp.float32),
        pltpu.SemaphoreType.DMA,
    ],
)
def _count_segments(seg_hbm, out_hbm, segv, onesv, zbuf, cnt_sh, sem):
    cid = lax.axis_index("c")
    sid = lax.axis_index("s")
    wid = sid * 2 + cid
    zeros16 = jnp.zeros((16,), jnp.float32)
    ones16 = jnp.ones((16,), jnp.float32)

    def _fill_z(j, carry):
        zbuf[j, :] = zeros16
        return carry

    lax.fori_loop(0, _ZB, _fill_z, 0)

    def _fill_o(j, carry):
        onesv[j, :] = ones16
        return carry

    lax.fori_loop(0, _CH, _fill_o, 0)

    for k in range((_NBLK + 15) // 16):
        b = sid + 16 * k

        @pl.when(b < _NBLK)
        def _():
            for m in range(_BLK // _ZB):
                pltpu.sync_copy(zbuf, cnt_sh.at[pl.ds(b * _BLK + m * _ZB, _ZB)])

    plsc.subcore_barrier()

    pltpu.sync_copy(seg_hbm.at[pl.ds(wid * _NCH, _NCH)], segv)

    def _scatter(j, carry):
        pltpu.sync_copy(onesv, cnt_sh.at[segv.at[j]], add=True)
        return carry

    lax.fori_loop(0, _NCH, _scatter, 0)
    plsc.subcore_barrier()

    for k in range((_NBLK + 15) // 16):
        b = sid + 16 * k

        @pl.when(b < _NBLK)
        def _():
            pltpu.sync_copy(cnt_sh.at[pl.ds(b * _BLK, _BLK)],
                            out_hbm.at[cid, pl.ds(b * _BLK, _BLK), :])


@functools.partial(
    pl.kernel,
    mesh=_sc_mesh(),
    out_type=jax.ShapeDtypeStruct((2, _SEG, _D), jnp.float32),
    compiler_params=pltpu.CompilerParams(use_tc_tiling_on_sc=False),
    scratch_types=[
        pltpu.VMEM((_QCH, _CH), jnp.int32),
        pltpu.VMEM((_QCH, _CH), jnp.int32),
        pltpu.VMEM((_NSLOT, _CH, _D), jnp.float32),
        pltpu.VMEM((_ZB, _D), jnp.float32),
        pltpu.VMEM_SHARED((_SEG, _D), jnp.float32),
        pltpu.SemaphoreType.DMA,
        pltpu.SemaphoreType.DMA,
        pltpu.SemaphoreType.DMA,
        pltpu.SemaphoreType.DMA,
    ],
)
def _segment_accumulate(hr_hbm, gidx_hbm, seg_hbm, cnt_hbm, out_hbm, gv, sv,
                        rows, zbuf, acc_sh, sg0, sg1, sg2, sg3):
    del cnt_hbm
    sems = [sg0, sg1, sg2, sg3]
    cid = lax.axis_index("c")
    sid = lax.axis_index("s")
    wid = sid * 2 + cid
    zeros16 = jnp.zeros((16,), jnp.float32)

    def _fill_z(j, carry):
        zbuf[j, pl.ds(0, 16)] = zeros16
        zbuf[j, pl.ds(16, 16)] = zeros16
        return carry

    lax.fori_loop(0, _ZB, _fill_z, 0)

    for k in range((_NBLK + 15) // 16):
        b = sid + 16 * k

        @pl.when(b < _NBLK)
        def _():
            for m in range(_BLK // _ZB):
                pltpu.sync_copy(zbuf, acc_sh.at[pl.ds(b * _BLK + m * _ZB, _ZB)])

    plsc.subcore_barrier()

    for q in range(4):
        pltpu.sync_copy(gidx_hbm.at[pl.ds(wid * _NCH + q * _QCH, _QCH)], gv)
        pltpu.sync_copy(seg_hbm.at[pl.ds(wid * _NCH + q * _QCH, _QCH)], sv)

        def _group(t, carry):
            handles = []
            for b in range(_NSLOT):
                jj = t * _NSLOT + b
                handles.append(pltpu.async_copy(
                    hr_hbm.at[gv.at[jj]], rows.at[b], sems[b]))
            for b in range(_NSLOT):
                jj = t * _NSLOT + b
                handles[b].wait()
                pltpu.sync_copy(rows.at[b], acc_sh.at[sv.at[jj]], add=True)
            return carry

        lax.fori_loop(0, _QCH // _NSLOT, _group, 0)
    plsc.subcore_barrier()

    for k in range((_NBLK + 15) // 16):
        b = sid + 16 * k

        @pl.when(b < _NBLK)
        def _():
            pltpu.sync_copy(acc_sh.at[pl.ds(b * _BLK, _BLK)],
                            out_hbm.at[cid, pl.ds(b * _BLK, _BLK), :])


def _prep_body(src_ref, dst_ref, et_ref, gidx_ref, seg_ref):
    et = et_ref[...]
    gidx_ref[...] = src_ref[...] * _R + et
    seg_ref[...] = dst_ref[...] * _R + et


def _icnt_body(cnt_ref, out_ref):
    c0 = cnt_ref[0, :, 0:1]
    c1 = cnt_ref[1, :, 0:1]
    out_ref[...] = 1.0 / jnp.maximum(c0 + c1, 1.0)


def _wcat(basis_ref, comp_ref):
    c = comp_ref[...]
    b0 = basis_ref[0]
    b1 = basis_ref[1]
    cols = [b0 * c[r:r + 1, 0:1] + b1 * c[r:r + 1, 1:2] for r in range(_R)]
    return jnp.concatenate(cols, axis=1)


def _hr0_body(x_ref, basis_ref, comp_ref, out_ref):
    w = _wcat(basis_ref, comp_ref)
    out_ref[...] = jnp.dot(x_ref[...], w, preferred_element_type=jnp.float32)


def _comb_core(s_ref, icnt_ref, h_ref, root_ref, bias_ref):
    s = s_ref[0] + s_ref[1]
    agg = s[:, 0:_D] * icnt_ref[:, 0:1]
    for r in range(1, _R):
        agg = agg + s[:, _D * r:_D * (r + 1)] * icnt_ref[:, r:r + 1]
    root_term = jnp.dot(h_ref[...], root_ref[...],
                        preferred_element_type=jnp.float32)
    return jnp.tanh(agg + root_term + bias_ref[...])


def _comb_mid_body(s_ref, icnt_ref, h_ref, root_ref, bias_ref,
                   basis_ref, comp_ref, hout_ref, hrn_ref):
    h = _comb_core(s_ref, icnt_ref, h_ref, root_ref, bias_ref)
    hout_ref[...] = h
    w = _wcat(basis_ref, comp_ref)
    hrn_ref[...] = jnp.dot(h, w, preferred_element_type=jnp.float32)


def _comb_last_body(s_ref, icnt_ref, h_ref, root_ref, bias_ref, hout_ref):
    hout_ref[...] = _comb_core(s_ref, icnt_ref, h_ref, root_ref, bias_ref)


def _final_body(h0_ref, h1_ref, h2_ref, h3_ref, bat_ref,
                w1_ref, b1_ref, w2_ref, b2_ref, out_ref, acc_ref):
    i = pl.program_id(0)

    @pl.when(i == 0)
    def _():
        acc_ref[...] = jnp.zeros_like(acc_ref)

    concat = jnp.concatenate(
        [h0_ref[...], h1_ref[...], h2_ref[...], h3_ref[...]], axis=1)
    g = lax.broadcasted_iota(jnp.int32, (128, _NB), 0)
    onehot = (bat_ref[0] == g).astype(jnp.float32)
    acc_ref[...] += jnp.dot(onehot, concat, preferred_element_type=jnp.float32)

    @pl.when(i == pl.num_programs(0) - 1)
    def _():
        pooled = acc_ref[...]
        hid = jnp.maximum(
            jnp.dot(pooled, w1_ref[...], preferred_element_type=jnp.float32)
            + b1_ref[...], 0.0)
        o = (jnp.dot(hid, w2_ref[...], preferred_element_type=jnp.float32)
             + b2_ref[...])
        m = jnp.max(o, axis=1, keepdims=True)
        lse = jnp.log(jnp.sum(jnp.exp(o - m), axis=1, keepdims=True)) + m
        out_ref[...] = o - lse


def kernel(x, edge_index, edge_type, batch,
           basis0, comp0, root0, bias0,
           basis1, comp1, root1, bias1,
           basis2, comp2, root2, bias2,
           basis3, comp3, root3, bias3,
           lin1_w, lin1_b, lin2_w, lin2_b):
    f32 = jnp.float32
    src2 = edge_index[0].reshape(2500, 128)
    dst2 = edge_index[1].reshape(2500, 128)
    et2 = edge_type.reshape(2500, 128)

    gidx2, seg2 = pl.pallas_call(
        _prep_body,
        out_shape=[jax.ShapeDtypeStruct((2500, 128), jnp.int32)] * 2,
    )(src2, dst2, et2)
    gidx = gidx2.reshape(_NW * _NCH, _CH)
    seg = seg2.reshape(_NW * _NCH, _CH)

    cnt2 = _count_segments(seg)
    icnt = (1.0 / jnp.maximum(cnt2[0, :, 0] + cnt2[1, :, 0], 1.0)
            ).reshape(_N, _R)

    hr = pl.pallas_call(
        _hr0_body,
        grid=(_GN,),
        in_specs=[
            pl.BlockSpec((_NB, 128), lambda i: (i, 0)),
            pl.BlockSpec((2, 128, _D), lambda i: (0, 0, 0)),
            pl.BlockSpec((_R, 2), lambda i: (0, 0)),
        ],
        out_specs=pl.BlockSpec((_NB, _R * _D), lambda i: (i, 0)),
        out_shape=jax.ShapeDtypeStruct((_N, _R * _D), f32),
    )(x, basis0, comp0)

    layers = [
        (root0, bias0, basis1, comp1),
        (root1, bias1, basis2, comp2),
        (root2, bias2, basis3, comp3),
        (root3, bias3, None, None),
    ]
    h_in = x
    states = []
    for l, (root_l, bias_l, bnext, cnext) in enumerate(layers):
        s2 = _segment_accumulate(hr.reshape(_SEG, _D), gidx, seg, cnt2)
        s2 = s2.reshape(2, _N, _R * _D)
        bias2d = bias_l.reshape(1, _D)
        din = h_in.shape[1]
        base_specs = [
            pl.BlockSpec((2, _NB, _R * _D), lambda i: (0, i, 0)),
            pl.BlockSpec((_NB, _R), lambda i: (i, 0)),
            pl.BlockSpec((_NB, din), lambda i: (i, 0)),
            pl.BlockSpec((din, _D), lambda i: (0, 0)),
            pl.BlockSpec((1, _D), lambda i: (0, 0)),
        ]
        if bnext is not None:
            h, hr = pl.pallas_call(
                _comb_mid_body,
                grid=(_GN,),
                in_specs=base_specs + [
                    pl.BlockSpec((2, _D, _D), lambda i: (0, 0, 0)),
                    pl.BlockSpec((_R, 2), lambda i: (0, 0)),
                ],
                out_specs=[
                    pl.BlockSpec((_NB, _D), lambda i: (i, 0)),
                    pl.BlockSpec((_NB, _R * _D), lambda i: (i, 0)),
                ],
                out_shape=[
                    jax.ShapeDtypeStruct((_N, _D), f32),
                    jax.ShapeDtypeStruct((_N, _R * _D), f32),
                ],
            )(s2, icnt, h_in, root_l, bias2d, bnext, cnext)
        else:
            h = pl.pallas_call(
                _comb_last_body,
                grid=(_GN,),
                in_specs=base_specs,
                out_specs=pl.BlockSpec((_NB, _D), lambda i: (i, 0)),
                out_shape=jax.ShapeDtypeStruct((_N, _D), f32),
            )(s2, icnt, h_in, root_l, bias2d)
        states.append(h)
        h_in = h

    hspec = pl.BlockSpec((_NB, _D), lambda i: (i, 0))
    out = pl.pallas_call(
        _final_body,
        grid=(_GN,),
        in_specs=[hspec, hspec, hspec, hspec,
                  pl.BlockSpec((1, 1, _NB), lambda i: (i, 0, 0)),
                  pl.BlockSpec((128, 128), lambda i: (0, 0)),
                  pl.BlockSpec((1, 128), lambda i: (0, 0)),
                  pl.BlockSpec((128, 5), lambda i: (0, 0)),
                  pl.BlockSpec((1, 5), lambda i: (0, 0))],
        out_specs=pl.BlockSpec((128, 5), lambda i: (0, 0)),
        out_shape=jax.ShapeDtypeStruct((128, 5), f32),
        scratch_shapes=[pltpu.VMEM((128, 128), f32)],
    )(states[0], states[1], states[2], states[3], batch.reshape(_GN, 1, _NB),
      lin1_w, lin1_b.reshape(1, 128), lin2_w, lin2_b.reshape(1, 5))
    return out
